# Initial kernel scaffold; baseline (speedup 1.0000x reference)
import jax
import jax.numpy as jnp
from jax.experimental import pallas as pl


def kernel(x, edge_index, Wp, bp, W0, b0, g0, be0, W1, b1, g1, be1, W2, b2, g2, be2, Wpool0, bpool0, Wpool1, bpool1, Wc0, bc0, Wc1, bc1):
    raise NotImplementedError("write your pallas kernel here")


# speedup vs baseline: 7.2828x; 7.2828x over previous
import functools

import jax
import jax.numpy as jnp
from jax import lax
from jax.experimental import pallas as pl
from jax.experimental.pallas import tpu as pltpu
from jax.experimental.pallas import tpu_sc as plsc

N = 10000
E = 320000
H = 128

NC = 2
NS = 16
NW = NC * NS
CHUNK = 128
CH = (E + NW * CHUNK - 1) // (NW * CHUNK)
EPT = CH * CHUNK
EPAD = NW * EPT
RACC = 10112
TRASH = N
RPT = RACC // NS

_mesh = plsc.VectorSubcoreMesh(core_axis_name="c", subcore_axis_name="s")


def _agg_body(v_hbm, gidx_hbm, sidx_hbm, zero_hbm, out_hbm,
              gidx_v, sidx_v, gbuf, acc, sem):
    c = lax.axis_index("c")
    s = lax.axis_index("s")
    wid = c * NS + s
    base = s * RPT
    pltpu.sync_copy(zero_hbm.at[pl.ds(base, RPT)], acc.at[pl.ds(base, RPT)])
    pltpu.sync_copy(gidx_hbm.at[wid], gidx_v)
    pltpu.sync_copy(sidx_hbm.at[wid], sidx_v)
    plsc.subcore_barrier()

    def body(j, carry):
        pltpu.async_copy(v_hbm.at[gidx_v.at[j]], gbuf, sem).wait()
        pltpu.sync_copy(gbuf, acc.at[sidx_v.at[j]], add=True)
        return carry

    lax.fori_loop(0, CH, body, 0)
    plsc.subcore_barrier()
    pltpu.sync_copy(acc.at[pl.ds(base, RPT)], out_hbm.at[c, pl.ds(base, RPT)])


_agg = pl.kernel(
    _agg_body,
    out_type=jax.ShapeDtypeStruct((NC, RACC, H), jnp.float32),
    mesh=_mesh,
    scratch_types=[
        pltpu.VMEM((CH, CHUNK), jnp.int32),
        pltpu.VMEM((CH, CHUNK), jnp.int32),
        pltpu.VMEM((CHUNK, H), jnp.float32),
        pltpu.VMEM_SHARED((RACC, H), jnp.float32),
        pltpu.SemaphoreType.DMA,
    ],
)


def _hist_body(s1_hbm, s2_hbm, outb_hbm, outd_hbm, i1_v, i2_v, hb, hd):
    c = lax.axis_index("c")
    s = lax.axis_index("s")
    wid = c * NS + s
    pltpu.sync_copy(s1_hbm.at[wid], i1_v)
    pltpu.sync_copy(s2_hbm.at[wid], i2_v)

    zeros = jnp.zeros((16,), jnp.float32)

    def zbody(i, carry):
        hb[pl.ds(i * 16, 16)] = zeros
        hd[pl.ds(i * 16, 16)] = zeros
        return carry

    lax.fori_loop(0, RACC // 16, zbody, 0)

    ones = jnp.ones((16,), jnp.float32)

    def ebody(i, carry):
        plsc.addupdate_scatter(hb, [i1_v[pl.ds(i * 16, 16)]], ones)
        plsc.addupdate_scatter(hd, [i2_v[pl.ds(i * 16, 16)]], ones)
        return carry

    lax.fori_loop(0, EPT // 16, ebody, 0)
    pltpu.sync_copy(hb, outb_hbm.at[wid])
    pltpu.sync_copy(hd, outd_hbm.at[wid])


_hist = pl.kernel(
    _hist_body,
    out_type=[
        jax.ShapeDtypeStruct((NW, RACC), jnp.float32),
        jax.ShapeDtypeStruct((NW, RACC), jnp.float32),
    ],
    mesh=_mesh,
    scratch_types=[
        pltpu.VMEM((EPT,), jnp.int32),
        pltpu.VMEM((EPT,), jnp.int32),
        pltpu.VMEM((RACC,), jnp.float32),
        pltpu.VMEM((RACC,), jnp.float32),
    ],
    compiler_params=pltpu.CompilerParams(needs_layout_passes=False),
)


BR = 1000
GR = N // BR


def _full(shape):
    nd = len(shape)
    return pl.BlockSpec(shape, lambda i, _nd=nd: (0,) * _nd)


def _prep_body(x_ref, wp_ref, bp_ref, w0_ref, h0_ref, hw_ref):
    h = jnp.dot(x_ref[...], wp_ref[...], preferred_element_type=jnp.float32, precision=lax.Precision.HIGHEST)
    h = jnp.maximum(h + bp_ref[...], 0.0)
    h0_ref[...] = h
    hw_ref[...] = jnp.dot(h, w0_ref[...], preferred_element_type=jnp.float32, precision=lax.Precision.HIGHEST)


_prep = pl.pallas_call(
    _prep_body,
    grid=(GR,),
    in_specs=[
        pl.BlockSpec((BR, H), lambda i: (i, 0)),
        _full((H, H)),
        _full((1, H)),
        _full((H, H)),
    ],
    out_specs=[pl.BlockSpec((BR, H), lambda i: (i, 0))] * 2,
    out_shape=[jax.ShapeDtypeStruct((N, H), jnp.float32)] * 2,
)


def _deg_body(hd_ref, hb_ref, dinv_ref, binv_ref):
    dn = jnp.sum(hd_ref[...], axis=0)
    dinv_ref[...] = jnp.where(dn > 0, 1.0 / dn, 0.0)
    bn = jnp.sum(hb_ref[...], axis=0)
    binv_ref[...] = jnp.where(bn > 0, 1.0 / bn, 0.0)


_deg = pl.pallas_call(
    _deg_body,
    out_shape=[jax.ShapeDtypeStruct((RACC,), jnp.float32)] * 2,
)


def _scale_body(p_ref, binv_ref, m_ref):
    m_ref[...] = (p_ref[0] + p_ref[1]) * binv_ref[...]


_scale = pl.pallas_call(
    _scale_body,
    grid=(GR,),
    in_specs=[
        pl.BlockSpec((NC, BR, H), lambda i: (0, i, 0)),
        pl.BlockSpec((BR, 1), lambda i: (i, 0)),
    ],
    out_specs=pl.BlockSpec((BR, H), lambda i: (i, 0)),
    out_shape=jax.ShapeDtypeStruct((N, H), jnp.float32),
)


def _norm_relu(p_ref, dinv_ref, b_ref, g_ref, be_ref):
    t = (p_ref[0] + p_ref[1]) * dinv_ref[...] + b_ref[...]
    mu = jnp.mean(t, axis=1, keepdims=True)
    d = t - mu
    var = jnp.mean(d * d, axis=1, keepdims=True)
    t = d * lax.rsqrt(var + 1e-5) * g_ref[...] + be_ref[...]
    return jnp.maximum(t, 0.0)


def _layer_body(p_ref, dinv_ref, b_ref, g_ref, be_ref, hres_ref, w_ref,
                h_ref, hw_ref, *, has_res):
    t = _norm_relu(p_ref, dinv_ref, b_ref, g_ref, be_ref)
    if has_res:
        t = t + hres_ref[...]
    h_ref[...] = t
    hw_ref[...] = jnp.dot(t, w_ref[...], preferred_element_type=jnp.float32, precision=lax.Precision.HIGHEST)


def _make_layer(has_res):
    return pl.pallas_call(
        functools.partial(_layer_body, has_res=has_res),
        grid=(GR,),
        in_specs=[
            pl.BlockSpec((NC, BR, H), lambda i: (0, i, 0)),
            pl.BlockSpec((BR, 1), lambda i: (i, 0)),
            _full((1, H)),
            _full((1, H)),
            _full((1, H)),
            pl.BlockSpec((BR, H), lambda i: (i, 0)),
            _full((H, H)),
        ],
        out_specs=[pl.BlockSpec((BR, H), lambda i: (i, 0))] * 2,
        out_shape=[jax.ShapeDtypeStruct((N, H), jnp.float32)] * 2,
    )


_layer0 = _make_layer(False)
_layer1 = _make_layer(True)


def _last_body(p_ref, dinv_ref, b_ref, g_ref, be_ref, hres_ref, h0_ref,
               wp0_ref, bp0_ref, wp1_ref, bp1_ref, wc0_ref, bc0_ref,
               wc1_ref, bc1_ref, out_ref):
    t = _norm_relu(p_ref, dinv_ref, b_ref, g_ref, be_ref)
    t = t + hres_ref[...] + h0_ref[...]
    wps = wp0_ref[0:H, :] + wp0_ref[H:2 * H, :]
    q = jnp.dot(t, wps, preferred_element_type=jnp.float32, precision=lax.Precision.HIGHEST) + bp0_ref[...]
    q = jnp.dot(q, wp1_ref[...], preferred_element_type=jnp.float32, precision=lax.Precision.HIGHEST) + bp1_ref[...]
    q = jnp.dot(q, wc0_ref[...], preferred_element_type=jnp.float32, precision=lax.Precision.HIGHEST) + bc0_ref[...]
    q = jnp.maximum(q, 0.0)
    out_ref[...] = jnp.dot(q, wc1_ref[...], preferred_element_type=jnp.float32, precision=lax.Precision.HIGHEST) + bc1_ref[...]


_last = pl.pallas_call(
    _last_body,
    grid=(GR,),
    in_specs=[
        pl.BlockSpec((NC, BR, H), lambda i: (0, i, 0)),
        pl.BlockSpec((BR, 1), lambda i: (i, 0)),
        _full((1, H)),
        _full((1, H)),
        _full((1, H)),
        pl.BlockSpec((BR, H), lambda i: (i, 0)),
        pl.BlockSpec((BR, H), lambda i: (i, 0)),
        _full((2 * H, H)),
        _full((1, H)),
        _full((H, H // 2)),
        _full((1, H // 2)),
        _full((H // 2, H // 4)),
        _full((1, H // 4)),
        _full((H // 4, 2)),
        _full((1, 2)),
    ],
    out_specs=pl.BlockSpec((BR, 2), lambda i: (i, 0)),
    out_shape=jax.ShapeDtypeStruct((N, 2), jnp.float32),
)


def kernel(x, edge_index, Wp, bp, W0, b0, g0, be0, W1, b1, g1, be1,
           W2, b2, g2, be2, Wpool0, bpool0, Wpool1, bpool1,
           Wc0, bc0, Wc1, bc1):
    ei0 = edge_index[0]
    ei1 = edge_index[1]
    pad_g = jnp.zeros((EPAD - E,), jnp.int32)
    pad_s = jnp.full((EPAD - E,), TRASH, jnp.int32)
    g1i = jnp.concatenate([ei0, pad_g]).reshape(NW, CH, CHUNK)
    s1i = jnp.concatenate([ei1, pad_s]).reshape(NW, CH, CHUNK)
    g2i = jnp.concatenate([ei1, pad_g]).reshape(NW, CH, CHUNK)
    s2i = jnp.concatenate([ei0, pad_s]).reshape(NW, CH, CHUNK)
    s1f = s1i.reshape(NW, EPT)
    s2f = s2i.reshape(NW, EPT)
    zeros_acc = jnp.zeros((RACC, H), jnp.float32)

    histB, histD = _hist(s1f, s2f)
    dinv1, binv1 = _deg(histD, histB)
    dinv = dinv1.reshape(RACC, 1)
    binv = binv1.reshape(RACC, 1)

    bp2 = bp.reshape(1, H)
    h, hw = _prep(x, Wp, bp2, W0)
    h0 = h

    Ws = [W0, W1, W2]
    bs = [b0.reshape(1, H), b1.reshape(1, H), b2.reshape(1, H)]
    gs = [g0.reshape(1, H), g1.reshape(1, H), g2.reshape(1, H)]
    bes = [be0.reshape(1, H), be1.reshape(1, H), be2.reshape(1, H)]

    for i in range(3):
        mp = _agg(hw, g1i, s1i, zeros_acc)
        m = _scale(mp, binv)
        op = _agg(m, g2i, s2i, zeros_acc)
        if i == 0:
            h, hw = _layer0(op, dinv, bs[i], gs[i], bes[i], h, Ws[i + 1])
        elif i == 1:
            h, hw = _layer1(op, dinv, bs[i], gs[i], bes[i], h, Ws[i + 1])
        else:
            out = _last(op, dinv, bs[i], gs[i], bes[i], h, h0,
                        Wpool0, bpool0.reshape(1, H),
                        Wpool1, bpool1.reshape(1, H // 2),
                        Wc0, bc0.reshape(1, H // 4),
                        Wc1, bc1.reshape(1, 2))
    return out
